# baseline (device time: 34519 ns/iter reference)
import jax
import jax.numpy as jnp
from jax import lax
from jax.experimental import pallas as pl
from jax.experimental.pallas import tpu as pltpu

N_DEV = 32


def kernel(x, w_mat):
    m_per, k = x.shape
    _, n = w_mat.shape
    n_per = n // N_DEV
    m_glob = N_DEV * m_per

    def body(x_ref, w_ref, out_ref, comm_ref, send_sems, recv_sems):
        me = lax.axis_index("i")

        y = jnp.dot(x_ref[:, :], w_ref[:, :], preferred_element_type=jnp.float32)

        for j in range(N_DEV):
            comm_ref[j] = y[:, j * n_per:(j + 1) * n_per]

        out_ref[pl.ds(me * m_per, m_per), :] = comm_ref[me]

        def send_desc(peer):
            return pltpu.make_async_remote_copy(
                src_ref=comm_ref.at[peer],
                dst_ref=out_ref.at[pl.ds(me * m_per, m_per), :],
                send_sem=send_sems.at[peer],
                recv_sem=recv_sems.at[me],
                device_id=(peer,),
                device_id_type=pl.DeviceIdType.MESH,
            )

        def recv_desc(src):
            return pltpu.make_async_remote_copy(
                src_ref=comm_ref.at[src],
                dst_ref=out_ref.at[pl.ds(src * m_per, m_per), :],
                send_sem=send_sems.at[src],
                recv_sem=recv_sems.at[src],
                device_id=(src,),
                device_id_type=pl.DeviceIdType.MESH,
            )

        for d in range(1, N_DEV):
            peer = lax.rem(me + d, N_DEV)
            send_desc(peer).start()

        for d in range(1, N_DEV):
            src = lax.rem(me + d, N_DEV)
            recv_desc(src).wait_recv()

        for d in range(1, N_DEV):
            peer = lax.rem(me + d, N_DEV)
            send_desc(peer).wait_send()

    return pl.pallas_call(
        body,
        out_shape=jax.ShapeDtypeStruct((m_glob, n_per), jnp.float32),
        in_specs=[
            pl.BlockSpec(memory_space=pltpu.VMEM),
            pl.BlockSpec(memory_space=pltpu.VMEM),
        ],
        out_specs=pl.BlockSpec(memory_space=pltpu.VMEM),
        scratch_shapes=[
            pltpu.VMEM((N_DEV, m_per, n_per), jnp.float32),
            pltpu.SemaphoreType.DMA((N_DEV,)),
            pltpu.SemaphoreType.DMA((N_DEV,)),
        ],
    )(x, w_mat)


# device time: 26689 ns/iter; 1.2934x vs baseline; 1.2934x over previous
import jax
import jax.numpy as jnp
from jax import lax
from jax.experimental import pallas as pl
from jax.experimental.pallas import tpu as pltpu

N_DEV = 32
B = 4


def kernel(x, w_mat):
    m_per, k = x.shape
    _, n = w_mat.shape
    n_per = n // N_DEV
    m_glob = N_DEV * m_per
    nb = n // B
    ppb = N_DEV // B

    def body(x_ref, w_hbm, out_ref, w_buf, comm_ref, load_sems, send_sems, recv_sems):
        me = lax.axis_index("i")

        barrier = pltpu.get_barrier_semaphore()
        for d in range(1, N_DEV):
            peer = lax.rem(me + d, N_DEV)
            pl.semaphore_signal(
                barrier, inc=1,
                device_id=(peer,), device_id_type=pl.DeviceIdType.MESH,
            )

        def load(b, slot):
            return pltpu.make_async_copy(
                w_hbm.at[:, pl.ds(b * nb, nb)],
                w_buf.at[slot],
                load_sems.at[slot],
            )

        load(0, 0).start()
        load(1, 1).start()

        x_val = x_ref[:, :]

        def send_desc(j):
            return pltpu.make_async_remote_copy(
                src_ref=comm_ref.at[j],
                dst_ref=out_ref.at[pl.ds(me * m_per, m_per), :],
                send_sem=send_sems.at[j],
                recv_sem=recv_sems.at[me],
                device_id=(j,),
                device_id_type=pl.DeviceIdType.MESH,
            )

        for b in range(B):
            slot = b % 2
            load(b, slot).wait()
            y_b = jnp.dot(
                x_val, w_buf[slot], preferred_element_type=jnp.float32
            )
            if b + 2 < B:
                load(b + 2, slot).start()
            for i in range(ppb):
                j = b * ppb + i
                comm_ref[j] = y_b[:, i * n_per:(i + 1) * n_per]
            if b == 0:
                pl.semaphore_wait(barrier, N_DEV - 1)
            for i in range(ppb):
                j = b * ppb + i

                @pl.when(j != me)
                def _():
                    send_desc(j).start()

        out_ref[pl.ds(me * m_per, m_per), :] = comm_ref[me]

        for d in range(1, N_DEV):
            src = lax.rem(me + d, N_DEV)
            pltpu.make_async_remote_copy(
                src_ref=comm_ref.at[src],
                dst_ref=out_ref.at[pl.ds(src * m_per, m_per), :],
                send_sem=send_sems.at[src],
                recv_sem=recv_sems.at[src],
                device_id=(src,),
                device_id_type=pl.DeviceIdType.MESH,
            ).wait_recv()

        for j in range(N_DEV):

            @pl.when(j != me)
            def _():
                send_desc(j).wait_send()

    return pl.pallas_call(
        body,
        out_shape=jax.ShapeDtypeStruct((m_glob, n_per), jnp.float32),
        in_specs=[
            pl.BlockSpec(memory_space=pltpu.VMEM),
            pl.BlockSpec(memory_space=pl.ANY),
        ],
        out_specs=pl.BlockSpec(memory_space=pltpu.VMEM),
        scratch_shapes=[
            pltpu.VMEM((2, k, nb), jnp.float32),
            pltpu.VMEM((N_DEV, m_per, n_per), jnp.float32),
            pltpu.SemaphoreType.DMA((2,)),
            pltpu.SemaphoreType.DMA((N_DEV,)),
            pltpu.SemaphoreType.DMA((N_DEV,)),
        ],
        compiler_params=pltpu.CompilerParams(collective_id=0),
    )(x, w_mat)
